# baseline (speedup 1.0000x reference)
import jax
import jax.numpy as jnp
from jax.experimental import pallas as pl
from jax.experimental.pallas import tpu as pltpu

_BN_EPS = 4e-5


def _make_body(hw, c_in, h, w, m_total, bn_eps):
    half = hw // 2

    def _body(xa_ref, xb_ref, w_ref, g_ref, b_ref, o_ref,
              x_buf, s_ref, sq_ref, ws_ref, shift_ref):
        p = pl.program_id(0)
        i = pl.program_id(1)

        @pl.when(p == 0)
        def _stats():
            @pl.when(i == 0)
            def _init():
                s_ref[...] = jnp.zeros_like(s_ref)
                sq_ref[...] = jnp.zeros_like(sq_ref)

            qa = xa_ref[0].reshape(half, c_in).astype(jnp.bfloat16)
            qb = xb_ref[0].reshape(half, c_in).astype(jnp.bfloat16)
            x_buf[i, :half] = qa
            x_buf[i, half:] = qb
            wtb = w_ref[...].astype(jnp.bfloat16)
            ya = jnp.dot(qa, wtb, preferred_element_type=jnp.float32)
            yb = jnp.dot(qb, wtb, preferred_element_type=jnp.float32)
            ones = jnp.ones((8, half), jnp.float32)
            s_ref[...] += (
                jnp.dot(ones, ya, preferred_element_type=jnp.float32)
                + jnp.dot(ones, yb, preferred_element_type=jnp.float32))
            sq_ref[...] += (
                jnp.dot(ones, ya * ya, preferred_element_type=jnp.float32)
                + jnp.dot(ones, yb * yb, preferred_element_type=jnp.float32))

        @pl.when(p == 1)
        def _normalize():
            @pl.when(i == 0)
            def _fold():
                mean = s_ref[0:1] / m_total
                e2 = sq_ref[0:1] / m_total
                var = jnp.maximum(e2 - mean * mean, 0.0)
                inv_std = 1.0 / jnp.sqrt(var + bn_eps)
                scale = g_ref[...] * inv_std
                shift_ref[...] = b_ref[...] - mean * scale
                ws_ref[...] = (w_ref[...] * scale).astype(jnp.bfloat16)

            y = jnp.dot(x_buf[i], ws_ref[...],
                        preferred_element_type=jnp.float32)
            z = jnp.maximum(y + shift_ref[...], 0.0)
            o_ref[0] = z.astype(o_ref.dtype).reshape(h, w, -1)

    return _body


def kernel(x, conv_w, conv_b, gamma, beta):
    N, Cin, H, W = x.shape
    Cout = conv_w.shape[0]
    HW = H * W
    M = N * HW
    del conv_b

    xt = jnp.transpose(x, (0, 2, 3, 1))
    wt = conv_w.reshape(Cout, Cin).T.astype(jnp.float32)
    g2 = gamma.astype(jnp.float32).reshape(1, Cout)
    b2 = beta.astype(jnp.float32).reshape(1, Cout)

    xa_spec = pl.BlockSpec(
        (1, H // 2, W, Cin),
        lambda p, i: (jnp.where(p == 0, i, N - 1), 0, 0, 0))
    xb_spec = pl.BlockSpec(
        (1, H // 2, W, Cin),
        lambda p, i: (jnp.where(p == 0, i, N - 1), 1, 0, 0))
    o_spec = pl.BlockSpec(
        (1, H, W, Cout), lambda p, i: (jnp.where(p == 0, 0, i), 0, 0, 0))
    w_spec = pl.BlockSpec((Cin, Cout), lambda p, i: (0, 0))
    vec_spec = pl.BlockSpec((1, Cout), lambda p, i: (0, 0))

    cost = pl.CostEstimate(
        flops=4 * M * Cin * Cout + 7 * M * Cout,
        transcendentals=Cout,
        bytes_accessed=M * Cin * 4 + M * Cout * 4 + Cout * Cin * 4)

    out = pl.pallas_call(
        _make_body(HW, Cin, H, W, M, _BN_EPS),
        out_shape=jax.ShapeDtypeStruct((N, H, W, Cout), x.dtype),
        grid=(2, N),
        in_specs=[xa_spec, xb_spec, w_spec, vec_spec, vec_spec],
        out_specs=o_spec,
        scratch_shapes=[
            pltpu.VMEM((N, HW, Cin), jnp.bfloat16),
            pltpu.VMEM((8, Cout), jnp.float32),
            pltpu.VMEM((8, Cout), jnp.float32),
            pltpu.VMEM((Cin, Cout), jnp.bfloat16),
            pltpu.VMEM((1, Cout), jnp.float32),
        ],
        compiler_params=pltpu.CompilerParams(
            dimension_semantics=("arbitrary", "arbitrary"),
            vmem_limit_bytes=61 * 1024 * 1024),
        cost_estimate=cost,
    )(xt, xt, wt, g2, b2)

    return jnp.transpose(out, (0, 3, 1, 2))


# speedup vs baseline: 1.1061x; 1.1061x over previous
import jax
import jax.numpy as jnp
from jax.experimental import pallas as pl
from jax.experimental.pallas import tpu as pltpu

_BN_EPS = 4e-5


def _make_body(hw, c_in, h, w, m_total, bn_eps):
    half = hw // 2

    def _body(xa_ref, xb_ref, w_ref, g_ref, b_ref, o_ref,
              x_buf, s_ref, sq_ref, ws_ref, shift_ref):
        p = pl.program_id(0)
        i = pl.program_id(1)

        @pl.when(p == 0)
        def _stats():
            @pl.when(i == 0)
            def _init():
                s_ref[...] = jnp.zeros_like(s_ref)
                sq_ref[...] = jnp.zeros_like(sq_ref)

            qa = xa_ref[0].reshape(half, c_in).astype(jnp.bfloat16)
            qb = xb_ref[0].reshape(half, c_in).astype(jnp.bfloat16)
            x_buf[i, :half] = qa
            x_buf[i, half:] = qb
            wtb = w_ref[...].astype(jnp.bfloat16)
            ya = jnp.dot(qa, wtb, preferred_element_type=jnp.float32)
            yb = jnp.dot(qb, wtb, preferred_element_type=jnp.float32)
            s_ref[...] += (jnp.sum(ya, axis=0, keepdims=True)
                           + jnp.sum(yb, axis=0, keepdims=True))
            sq_ref[...] += (jnp.sum(ya * ya, axis=0, keepdims=True)
                            + jnp.sum(yb * yb, axis=0, keepdims=True))

        @pl.when(p == 1)
        def _normalize():
            @pl.when(i == 0)
            def _fold():
                mean = s_ref[...] / m_total
                e2 = sq_ref[...] / m_total
                var = jnp.maximum(e2 - mean * mean, 0.0)
                inv_std = 1.0 / jnp.sqrt(var + bn_eps)
                scale = g_ref[...] * inv_std
                shift_ref[...] = b_ref[...] - mean * scale
                ws_ref[...] = (w_ref[...] * scale).astype(jnp.bfloat16)

            y = jnp.dot(x_buf[i], ws_ref[...],
                        preferred_element_type=jnp.float32)
            z = jnp.maximum(y + shift_ref[...], 0.0)
            o_ref[0] = z.astype(o_ref.dtype).reshape(h, w, -1)

    return _body


def kernel(x, conv_w, conv_b, gamma, beta):
    N, Cin, H, W = x.shape
    Cout = conv_w.shape[0]
    HW = H * W
    M = N * HW
    del conv_b

    xt = jnp.transpose(x, (0, 2, 3, 1))
    wt = conv_w.reshape(Cout, Cin).T.astype(jnp.float32)
    g2 = gamma.astype(jnp.float32).reshape(1, Cout)
    b2 = beta.astype(jnp.float32).reshape(1, Cout)

    xa_spec = pl.BlockSpec(
        (1, H // 2, W, Cin),
        lambda p, i: (jnp.where(p == 0, i, N - 1), 0, 0, 0))
    xb_spec = pl.BlockSpec(
        (1, H // 2, W, Cin),
        lambda p, i: (jnp.where(p == 0, i, N - 1), 1, 0, 0))
    o_spec = pl.BlockSpec(
        (1, H, W, Cout), lambda p, i: (jnp.where(p == 0, 0, i), 0, 0, 0))
    w_spec = pl.BlockSpec((Cin, Cout), lambda p, i: (0, 0))
    vec_spec = pl.BlockSpec((1, Cout), lambda p, i: (0, 0))

    cost = pl.CostEstimate(
        flops=4 * M * Cin * Cout + 7 * M * Cout,
        transcendentals=Cout,
        bytes_accessed=M * Cin * 4 + M * Cout * 4 + Cout * Cin * 4)

    out = pl.pallas_call(
        _make_body(HW, Cin, H, W, M, _BN_EPS),
        out_shape=jax.ShapeDtypeStruct((N, H, W, Cout), x.dtype),
        grid=(2, N),
        in_specs=[xa_spec, xb_spec, w_spec, vec_spec, vec_spec],
        out_specs=o_spec,
        scratch_shapes=[
            pltpu.VMEM((N, HW, Cin), jnp.bfloat16),
            pltpu.VMEM((1, Cout), jnp.float32),
            pltpu.VMEM((1, Cout), jnp.float32),
            pltpu.VMEM((Cin, Cout), jnp.bfloat16),
            pltpu.VMEM((1, Cout), jnp.float32),
        ],
        compiler_params=pltpu.CompilerParams(
            dimension_semantics=("arbitrary", "arbitrary"),
            vmem_limit_bytes=61 * 1024 * 1024),
        cost_estimate=cost,
    )(xt, xt, wt, g2, b2)

    return jnp.transpose(out, (0, 3, 1, 2))
